# baseline (speedup 1.0000x reference)
import functools

import jax
import jax.numpy as jnp
from jax import lax
from jax.experimental import pallas as pl
from jax.experimental.pallas import tpu as pltpu
from jax.experimental.pallas import tpu_sc as plsc

_N = 4096
_P = 512
_NP = 8
_NPATCH = 64
_PW = _P // 4
_R = 64
_SLAB_PER_PATCH = _P // _R
_NBUF = 4
_NBIN = 256
_HPAD = 257
_NPIX = _P * _P

_MESH = plsc.VectorSubcoreMesh(core_axis_name="c", subcore_axis_name="s")


def _binize_abs_body(sr_ref, hr_ref, bins_ref, psum_ref):
    h = hr_ref[...]
    s = sr_ref[...]
    psum_ref[...] = jnp.sum(jnp.abs(s - h), axis=0).reshape(1, 1, _P)
    bins_ref[...] = (h * 255.0).astype(jnp.int32).astype(jnp.uint8).reshape(1, _P, _P)


def _tc_binize_abs(sr, hr):
    return pl.pallas_call(
        _binize_abs_body,
        grid=(_NPATCH,),
        in_specs=[
            pl.BlockSpec((_P, _P), lambda p: (p // _NP, p % _NP)),
            pl.BlockSpec((_P, _P), lambda p: (p // _NP, p % _NP)),
        ],
        out_specs=[
            pl.BlockSpec((1, _P, _P), lambda p: (p, 0, 0)),
            pl.BlockSpec((1, 1, _P), lambda p: (p, 0, 0)),
        ],
        out_shape=[
            jax.ShapeDtypeStruct((_NPATCH, _P, _P), jnp.uint8),
            jax.ShapeDtypeStruct((_NPATCH, 1, _P), jnp.float32),
        ],
    )(sr, hr)


@functools.partial(
    pl.kernel,
    out_type=jax.ShapeDtypeStruct((_NPATCH, 16 * _HPAD), jnp.float32),
    mesh=_MESH,
    compiler_params=pltpu.CompilerParams(needs_layout_passes=False),
    scratch_types=[
        pltpu.VMEM((_NBUF, _R, _PW), jnp.uint32),
        pltpu.VMEM((16 * _HPAD,), jnp.float32),
        pltpu.SemaphoreType.DMA,
        pltpu.SemaphoreType.DMA,
        pltpu.SemaphoreType.DMA,
        pltpu.SemaphoreType.DMA,
    ],
)
def _sc_hist(bins_hbm, hist_out, bv, hist_v, sem0, sem1, sem2, sem3):
    wid = lax.axis_index("s") * 2 + lax.axis_index("c")
    lane_base = lax.iota(jnp.int32, 16) * _HPAD
    ones = jnp.ones((16,), jnp.float32)
    zeros16 = jnp.zeros((16,), jnp.float32)
    sems = (sem0, sem1, sem2, sem3)
    m255 = jnp.uint32(255)

    def zero_hist():
        def zb(i, carry):
            hist_v[pl.ds(i * 16, 16)] = zeros16
            return carry
        lax.fori_loop(0, _HPAD, zb, 0)

    zero_hist()
    _U = 4
    _FLAT = _R * (_PW // 16)
    _VSH = 3
    _GROUPS = _SLAB_PER_PATCH // _NBUF

    def patch_src(p, t):
        return bins_hbm.at[p, pl.ds(t * _R, _R)]

    for pp in range(2):
        p = wid * 2 + pp
        for b in range(_NBUF):
            pltpu.async_copy(patch_src(p, b), bv.at[b], sems[b])

        def group_body(j, carry, p=p):
            for b in range(_NBUF):
                t = _NBUF * j + b
                pltpu.make_async_copy(patch_src(p, 0), bv.at[b], sems[b]).wait()

                def slab_body(i, c, b=b):
                    for u in range(_U):
                        v = i + u
                        r = v >> _VSH
                        k = v - (r << _VSH)
                        w = bv[b, r, pl.ds(k * 16, 16)]
                        b0 = (w & m255).astype(jnp.int32)
                        b1 = ((w >> 8) & m255).astype(jnp.int32)
                        b2 = ((w >> 16) & m255).astype(jnp.int32)
                        b3 = (w >> 24).astype(jnp.int32)
                        plsc.addupdate_scatter(hist_v, [lane_base + b0], ones)
                        plsc.addupdate_scatter(hist_v, [lane_base + b1], ones)
                        plsc.addupdate_scatter(hist_v, [lane_base + b2], ones)
                        plsc.addupdate_scatter(hist_v, [lane_base + b3], ones)
                    return c

                plsc.parallel_loop(0, _FLAT, _U, carry=jnp.int32(0))(slab_body)

                @pl.when(t + _NBUF < _SLAB_PER_PATCH)
                def _(p=p, t=t, b=b):
                    pltpu.async_copy(patch_src(p, t + _NBUF), bv.at[b], sems[b])
            return carry

        lax.fori_loop(0, _GROUPS, group_body, 0)
        pltpu.sync_copy(hist_v, hist_out.at[p])
        if pp == 0:
            zero_hist()


def _combine_body(hist_ref, psum_ref, out_ref):
    h = hist_ref[...]
    counts = jnp.sum(h, axis=1)[:, 0:_NBIN]
    prob = counts * (1.0 / _NPIX)
    pos = counts > 0.0
    logp = jnp.log(jnp.where(pos, prob, 1.0))
    terms = jnp.where(pos, prob * logp, 0.0) * (-1.0 / jnp.log(2.0))

    ent = jnp.sum(terms[:, 0:16], axis=1, keepdims=True)
    comp = jnp.zeros_like(ent)
    for g in range(1, 16):
        y = jnp.sum(terms[:, g * 16:(g + 1) * 16], axis=1, keepdims=True) - comp
        t = ent + y
        comp = (t - ent) - y
        ent = t

    emin = jnp.min(ent)
    emax = jnp.max(ent)
    w = (ent - emin) / emax
    s = jnp.sum(psum_ref[...], axis=1, keepdims=True)
    out_ref[...] = jnp.reshape(jnp.sum(w * s) * (1.0 / (_N * _N)), (1, 1))


def kernel(sr, hr):
    bins_u8, psum = _tc_binize_abs(sr, hr)
    bins32 = lax.bitcast_convert_type(
        bins_u8.reshape(_NPATCH, _P, _PW, 4), jnp.uint32)
    hist = _sc_hist(bins32)
    out = pl.pallas_call(
        _combine_body,
        out_shape=jax.ShapeDtypeStruct((1, 1), jnp.float32),
    )(hist.reshape(_NPATCH, 16, _HPAD), psum.reshape(_NPATCH, _P))
    return out[0, 0]


# speedup vs baseline: 1.8171x; 1.8171x over previous
import functools

import jax
import jax.numpy as jnp
from jax import lax
from jax.experimental import pallas as pl
from jax.experimental.pallas import tpu as pltpu
from jax.experimental.pallas import tpu_sc as plsc

_N = 4096
_P = 512
_NP = 8
_NPATCH = 64
_PW = _P // 4
_R = 64
_SLAB_PER_PATCH = _P // _R
_NBUF = 4
_NBIN = 256
_HPAD = 257
_NPIX = _P * _P

_MESH = plsc.VectorSubcoreMesh(core_axis_name="c", subcore_axis_name="s")


def _binize_abs_body(sr_ref, hr_ref, bins_ref, psum_ref):
    h = hr_ref[...]
    s = sr_ref[...]
    psum_ref[...] = jnp.sum(jnp.abs(s - h), axis=0).reshape(1, 1, _P)
    bins_ref[...] = (h * 255.0).astype(jnp.int32).astype(jnp.uint8).reshape(1, _P, _P)


def _tc_binize_abs(sr, hr):
    return pl.pallas_call(
        _binize_abs_body,
        grid=(_NPATCH,),
        in_specs=[
            pl.BlockSpec((_P, _P), lambda p: (p // _NP, p % _NP)),
            pl.BlockSpec((_P, _P), lambda p: (p // _NP, p % _NP)),
        ],
        out_specs=[
            pl.BlockSpec((1, _P, _P), lambda p: (p, 0, 0)),
            pl.BlockSpec((1, 1, _P), lambda p: (p, 0, 0)),
        ],
        out_shape=[
            jax.ShapeDtypeStruct((_NPATCH, _P, _P), jnp.uint8),
            jax.ShapeDtypeStruct((_NPATCH, 1, _P), jnp.float32),
        ],
    )(sr, hr)


@functools.partial(
    pl.kernel,
    out_type=jax.ShapeDtypeStruct((_NPATCH, 16 * _HPAD), jnp.float32),
    mesh=_MESH,
    compiler_params=pltpu.CompilerParams(needs_layout_passes=False),
    scratch_types=[
        pltpu.VMEM((_NBUF, _R, _P), jnp.uint8),
        pltpu.VMEM((16 * _HPAD,), jnp.float32),
        pltpu.SemaphoreType.DMA,
        pltpu.SemaphoreType.DMA,
        pltpu.SemaphoreType.DMA,
        pltpu.SemaphoreType.DMA,
    ],
)
def _sc_hist(bins_hbm, hist_out, bv, hist_v, sem0, sem1, sem2, sem3):
    wid = lax.axis_index("s") * 2 + lax.axis_index("c")
    lane_base = lax.iota(jnp.int32, 16) * _HPAD
    ones = jnp.ones((16,), jnp.float32)
    zeros16 = jnp.zeros((16,), jnp.float32)
    sems = (sem0, sem1, sem2, sem3)
    m255 = jnp.uint32(255)

    def zero_hist():
        def zb(i, carry):
            hist_v[pl.ds(i * 16, 16)] = zeros16
            return carry
        lax.fori_loop(0, _HPAD, zb, 0)

    zero_hist()
    _U = 4
    _FLAT = _R * (_PW // 16)
    _VSH = 3
    _GROUPS = _SLAB_PER_PATCH // _NBUF

    def patch_src(p, t):
        return bins_hbm.at[p, pl.ds(t * _R, _R)]

    for pp in range(2):
        p = wid * 2 + pp
        for b in range(_NBUF):
            pltpu.async_copy(patch_src(p, b), bv.at[b], sems[b])

        def group_body(j, carry, p=p):
            for b in range(_NBUF):
                t = _NBUF * j + b
                pltpu.make_async_copy(patch_src(p, 0), bv.at[b], sems[b]).wait()

                def slab_body(i, c, b=b):
                    for u in range(_U):
                        v = i + u
                        r = v >> _VSH
                        k = v - (r << _VSH)
                        w = plsc.bitcast(bv[b, r, pl.ds(k * 64, 64)], jnp.uint32)
                        b0 = (w & m255).astype(jnp.int32)
                        b1 = ((w >> 8) & m255).astype(jnp.int32)
                        b2 = ((w >> 16) & m255).astype(jnp.int32)
                        b3 = (w >> 24).astype(jnp.int32)
                        plsc.addupdate_scatter(hist_v, [lane_base + b0], ones)
                        plsc.addupdate_scatter(hist_v, [lane_base + b1], ones)
                        plsc.addupdate_scatter(hist_v, [lane_base + b2], ones)
                        plsc.addupdate_scatter(hist_v, [lane_base + b3], ones)
                    return c

                plsc.parallel_loop(0, _FLAT, _U, carry=jnp.int32(0))(slab_body)

                @pl.when(t + _NBUF < _SLAB_PER_PATCH)
                def _(p=p, t=t, b=b):
                    pltpu.async_copy(patch_src(p, t + _NBUF), bv.at[b], sems[b])
            return carry

        lax.fori_loop(0, _GROUPS, group_body, 0)
        pltpu.sync_copy(hist_v, hist_out.at[p])
        if pp == 0:
            zero_hist()


def _combine_body(hist_ref, psum_ref, out_ref):
    h = hist_ref[...]
    counts = jnp.sum(h, axis=1)[:, 0:_NBIN]
    prob = counts * (1.0 / _NPIX)
    pos = counts > 0.0
    logp = jnp.log(jnp.where(pos, prob, 1.0))
    terms = jnp.where(pos, prob * logp, 0.0) * (-1.0 / jnp.log(2.0))

    ent = jnp.sum(terms[:, 0:16], axis=1, keepdims=True)
    comp = jnp.zeros_like(ent)
    for g in range(1, 16):
        y = jnp.sum(terms[:, g * 16:(g + 1) * 16], axis=1, keepdims=True) - comp
        t = ent + y
        comp = (t - ent) - y
        ent = t

    emin = jnp.min(ent)
    emax = jnp.max(ent)
    w = (ent - emin) / emax
    s = jnp.sum(psum_ref[...], axis=1, keepdims=True)
    out_ref[...] = jnp.reshape(jnp.sum(w * s) * (1.0 / (_N * _N)), (1, 1))


def kernel(sr, hr):
    bins_u8, psum = _tc_binize_abs(sr, hr)
    hist = _sc_hist(bins_u8)
    out = pl.pallas_call(
        _combine_body,
        out_shape=jax.ShapeDtypeStruct((1, 1), jnp.float32),
    )(hist.reshape(_NPATCH, 16, _HPAD), psum.reshape(_NPATCH, _P))
    return out[0, 0]


# speedup vs baseline: 2.6346x; 1.4498x over previous
import functools

import jax
import jax.numpy as jnp
from jax import lax
from jax.experimental import pallas as pl
from jax.experimental.pallas import tpu as pltpu
from jax.experimental.pallas import tpu_sc as plsc

_N = 4096
_P = 512
_NP = 8
_NPATCH = 64
_R = 64
_SLAB_PER_PATCH = _P // _R
_NSLAB = 2 * _SLAB_PER_PATCH
_NBIN = 256
_HPAD = 257
_NPIX = _P * _P

_MESH = plsc.VectorSubcoreMesh(core_axis_name="c", subcore_axis_name="s")


@functools.partial(
    pl.kernel,
    out_type=jax.ShapeDtypeStruct((_NPATCH, 16 * _HPAD), jnp.float32),
    mesh=_MESH,
    compiler_params=pltpu.CompilerParams(needs_layout_passes=False),
    scratch_types=[
        pltpu.VMEM((2, _R, _P), jnp.float32),
        pltpu.VMEM((16 * _HPAD,), jnp.float32),
        pltpu.SemaphoreType.DMA,
        pltpu.SemaphoreType.DMA,
    ],
)
def _sc_hist(hr_hbm, hist_out, hr_v, hist_v, sem0, sem1):
    wid = lax.axis_index("s") * 2 + lax.axis_index("c")
    lane_base = lax.iota(jnp.int32, 16) * _HPAD
    ones = jnp.ones((16,), jnp.float32)
    zeros16 = jnp.zeros((16,), jnp.float32)
    sems = (sem0, sem1)

    def src(t):
        p = wid * 2 + t // _SLAB_PER_PATCH
        r0 = (p // _NP) * _P + (t % _SLAB_PER_PATCH) * _R
        c0 = (p % _NP) * _P
        return hr_hbm.at[pl.ds(r0, _R), pl.ds(c0, _P)]

    def zero_hist():
        def zb(i, carry):
            hist_v[pl.ds(i * 16, 16)] = zeros16
            return carry
        lax.fori_loop(0, _HPAD, zb, 0)

    zero_hist()
    _U = 8
    _FLAT = _R * (_P // 16)
    _VSH = 5
    _PAIRS = _SLAB_PER_PATCH // 2

    def patch_src(p, t):
        r0 = (p // _NP) * _P + t * _R
        c0 = (p % _NP) * _P
        return hr_hbm.at[pl.ds(r0, _R), pl.ds(c0, _P)]

    for pp in range(2):
        p = wid * 2 + pp
        pltpu.async_copy(patch_src(p, 0), hr_v.at[0], sems[0])
        pltpu.async_copy(patch_src(p, 1), hr_v.at[1], sems[1])

        def pair_body(j, carry, p=p):
            for b in range(2):
                t = 2 * j + b
                pltpu.make_async_copy(patch_src(p, 0), hr_v.at[b], sems[b]).wait()

                def slab_body(i, c, b=b):
                    for u in range(_U):
                        v = i + u
                        r = v >> _VSH
                        k = v - (r << _VSH)
                        h = hr_v[b, r, pl.ds(k * 16, 16)]
                        bin_i = (h * 255.0).astype(jnp.int32)
                        plsc.addupdate_scatter(hist_v, [lane_base + bin_i], ones)
                    return c

                plsc.parallel_loop(0, _FLAT, _U, carry=jnp.int32(0))(slab_body)

                @pl.when(t + 2 < _SLAB_PER_PATCH)
                def _(p=p, t=t, b=b):
                    pltpu.async_copy(patch_src(p, t + 2), hr_v.at[b], sems[b])
            return carry

        lax.fori_loop(0, _PAIRS, pair_body, 0)
        pltpu.sync_copy(hist_v, hist_out.at[p])
        if pp == 0:
            zero_hist()


def _abs_body(sr_ref, hr_ref, out_ref):
    d = jnp.abs(sr_ref[...] - hr_ref[...])
    out_ref[...] = jnp.sum(d, axis=0).reshape(1, 1, _P)


def _tc_abs(sr, hr):
    return pl.pallas_call(
        _abs_body,
        grid=(_NPATCH,),
        in_specs=[
            pl.BlockSpec((_P, _P), lambda p: (p // _NP, p % _NP)),
            pl.BlockSpec((_P, _P), lambda p: (p // _NP, p % _NP)),
        ],
        out_specs=pl.BlockSpec((1, 1, _P), lambda p: (p, 0, 0)),
        out_shape=jax.ShapeDtypeStruct((_NPATCH, 1, _P), jnp.float32),
    )(sr, hr)


def _combine_body(hist_ref, psum_ref, out_ref):
    h = hist_ref[...]
    counts = jnp.sum(h, axis=1)[:, 0:_NBIN]
    prob = counts * (1.0 / _NPIX)
    pos = counts > 0.0
    logp = jnp.log(jnp.where(pos, prob, 1.0))
    terms = jnp.where(pos, prob * logp, 0.0) * (-1.0 / jnp.log(2.0))

    ent = jnp.sum(terms[:, 0:16], axis=1, keepdims=True)
    comp = jnp.zeros_like(ent)
    for g in range(1, 16):
        y = jnp.sum(terms[:, g * 16:(g + 1) * 16], axis=1, keepdims=True) - comp
        t = ent + y
        comp = (t - ent) - y
        ent = t

    emin = jnp.min(ent)
    emax = jnp.max(ent)
    w = (ent - emin) / emax
    s = jnp.sum(psum_ref[...], axis=1, keepdims=True)
    out_ref[...] = jnp.reshape(jnp.sum(w * s) * (1.0 / (_N * _N)), (1, 1))


def kernel(sr, hr):
    psum = _tc_abs(sr, hr)
    hist = _sc_hist(hr)
    out = pl.pallas_call(
        _combine_body,
        out_shape=jax.ShapeDtypeStruct((1, 1), jnp.float32),
    )(hist.reshape(_NPATCH, 16, _HPAD), psum.reshape(_NPATCH, _P))
    return out[0, 0]
